# baseline (speedup 1.0000x reference)
import functools

import jax
import jax.numpy as jnp
from jax import lax
from jax.experimental import pallas as pl
from jax.experimental.pallas import tpu as pltpu
from jax.experimental.pallas import tpu_sc as plsc

_B = 16384
_E = 64


def _make_gather():
    info = plsc.get_sparse_core_info()
    nc, ns = info.num_cores, info.num_subcores
    nw = nc * ns
    bpw = _B // nw

    mesh = plsc.VectorSubcoreMesh(core_axis_name="c", subcore_axis_name="s")

    @functools.partial(
        pl.kernel,
        out_type=(
            jax.ShapeDtypeStruct((_B, 2 * _E), jnp.float32),
            jax.ShapeDtypeStruct((_B, 2 * _E), jnp.float32),
        ),
        mesh=mesh,
        scratch_types=[
            pltpu.VMEM((bpw,), jnp.int32),
            pltpu.VMEM((bpw, 2 * _E), jnp.float32),
            pltpu.SemaphoreType.DMA,
        ],
    )
    def gather(uid_hbm, sid_hbm, ut3, st3, ue_hbm, se_hbm,
               ids_v, rows_v, gsem):
        wid = lax.axis_index("s") * nc + lax.axis_index("c")
        base = wid * bpw

        def run_table(id_hbm, tab3, out_hbm):
            pltpu.sync_copy(id_hbm.at[pl.ds(base, bpw)], ids_v)

            def body(c, carry):
                vec = ids_v[pl.ds(c * 16, 16)]
                for k in range(16):
                    rid = vec[k]
                    tid = lax.shift_right_logical(rid, 4)
                    r = lax.shift_right_logical(rid, 1) & 7
                    pltpu.async_copy(
                        tab3.at[tid, pl.ds(r, 1)],
                        rows_v.at[pl.ds(c * 16 + k, 1)], gsem)
                return carry

            lax.fori_loop(0, bpw // 16, body, 0)
            pltpu.make_async_copy(
                out_hbm.at[pl.ds(0, bpw)], rows_v, gsem).wait()
            pltpu.sync_copy(rows_v, out_hbm.at[pl.ds(base, bpw)])

        run_table(uid_hbm, ut3, ue_hbm)
        run_table(sid_hbm, st3, se_hbm)

    return gather


_gather = _make_gather()


def _mlp_body(ue2_ref, se2_ref, uid_ref, sid_ref, w1u_ref, w1s_ref, b1_ref,
              w2_ref, b2_ref, w3_ref, b3_ref, wo_ref, bo_ref, out_ref):
    upar = (uid_ref[...] & 1) == 1
    spar = (sid_ref[...] & 1) == 1
    ue = jnp.where(upar, ue2_ref[:, _E:], ue2_ref[:, :_E])
    se = jnp.where(spar, se2_ref[:, _E:], se2_ref[:, :_E])
    x = jnp.dot(ue, w1u_ref[...], preferred_element_type=jnp.float32)
    x = x + jnp.dot(se, w1s_ref[...], preferred_element_type=jnp.float32)
    h = jnp.maximum(x + b1_ref[...], 0.0)
    h = jnp.maximum(
        jnp.dot(h, w2_ref[...], preferred_element_type=jnp.float32)
        + b2_ref[...], 0.0)
    h = jnp.maximum(
        jnp.dot(h, w3_ref[...], preferred_element_type=jnp.float32)
        + b3_ref[...], 0.0)
    o = jnp.sum(h * wo_ref[...], axis=1, keepdims=True) + bo_ref[...]
    out_ref[...] = 1.0 / (1.0 + jnp.exp(-o))


def _mlp(ue2, se2, uids, sids, w1u, w1s, b1, w2t, b2, w3t, b3, wo_row, bo):
    bn = 2048
    grid = (_B // bn,)
    full = lambda shape: pl.BlockSpec(shape, lambda i: (0, 0))
    return pl.pallas_call(
        _mlp_body,
        grid=grid,
        in_specs=[
            pl.BlockSpec((bn, 2 * _E), lambda i: (i, 0)),
            pl.BlockSpec((bn, 2 * _E), lambda i: (i, 0)),
            pl.BlockSpec((bn, 1), lambda i: (i, 0)),
            pl.BlockSpec((bn, 1), lambda i: (i, 0)),
            full((_E, 128)),
            full((_E, 128)),
            full((1, 128)),
            full((128, 64)),
            full((1, 64)),
            full((64, 32)),
            full((1, 32)),
            full((1, 32)),
            full((1, 1)),
        ],
        out_specs=pl.BlockSpec((bn, 1), lambda i: (i, 0)),
        out_shape=jax.ShapeDtypeStruct((_B, 1), jnp.float32),
    )(ue2, se2, uids, sids, w1u, w1s, b1, w2t, b2, w3t, b3, wo_row, bo)


def kernel(user_ids, symbol_ids, user_table, symbol_table,
           W1, b1, W2, b2, W3, b3, Wo, bo):
    uids = user_ids.astype(jnp.int32)
    sids = symbol_ids.astype(jnp.int32)
    ut3 = user_table.reshape(-1, 8, 2 * _E)
    st3 = symbol_table.reshape(-1, 8, 2 * _E)
    ue2, se2 = _gather(uids, sids, ut3, st3)
    w1t = W1.T
    return _mlp(ue2, se2, uids.reshape(-1, 1), sids.reshape(-1, 1),
                w1t[:_E], w1t[_E:], b1.reshape(1, -1),
                W2.T, b2.reshape(1, -1), W3.T, b3.reshape(1, -1),
                Wo.reshape(1, -1), bo.reshape(1, 1))


# speedup vs baseline: 2.4155x; 2.4155x over previous
import functools

import jax
import jax.numpy as jnp
from jax import lax
from jax.experimental import pallas as pl
from jax.experimental.pallas import tpu as pltpu
from jax.experimental.pallas import tpu_sc as plsc

_B = 16384
_E = 64


def _make_gather():
    info = plsc.get_sparse_core_info()
    nc, ns = info.num_cores, info.num_subcores
    nw = nc * ns
    bpw = _B // nw

    mesh = plsc.VectorSubcoreMesh(core_axis_name="c", subcore_axis_name="s")

    @functools.partial(
        pl.kernel,
        out_type=(
            jax.ShapeDtypeStruct((_B, _E), jnp.float32),
            jax.ShapeDtypeStruct((_B, _E), jnp.float32),
        ),
        mesh=mesh,
        scratch_types=[
            pltpu.VMEM((bpw,), jnp.int32),
            pltpu.VMEM((bpw, _E), jnp.float32),
            pltpu.SemaphoreType.DMA,
        ],
    )
    def gather(uid_hbm, sid_hbm, ut3, st3, ue_hbm, se_hbm,
               ids_v, rows_v, gsem):
        wid = lax.axis_index("s") * nc + lax.axis_index("c")
        base = wid * bpw

        def run_table(id_hbm, tab3, out_hbm):
            pltpu.sync_copy(id_hbm.at[pl.ds(base, bpw)], ids_v)

            def body(c, carry):
                vec = ids_v[pl.ds(c * 16, 16)]
                for k in range(16):
                    rid = vec[k]
                    tid = lax.shift_right_logical(rid, 3)
                    r = rid & 7
                    pltpu.async_copy(
                        tab3.at[tid, pl.ds(r, 1)],
                        rows_v.at[pl.ds(c * 16 + k, 1)], gsem)
                return carry

            lax.fori_loop(0, bpw // 16, body, 0)
            pltpu.make_async_copy(
                out_hbm.at[pl.ds(0, bpw)], rows_v, gsem).wait()
            pltpu.sync_copy(rows_v, out_hbm.at[pl.ds(base, bpw)])

        run_table(uid_hbm, ut3, ue_hbm)
        run_table(sid_hbm, st3, se_hbm)

    return gather


_gather = _make_gather()


def _mlp_body(ue_ref, se_ref, w1u_ref, w1s_ref, b1_ref, w2_ref, b2_ref,
              w3_ref, b3_ref, wo_ref, bo_ref, out_ref):
    x = jnp.dot(ue_ref[...], w1u_ref[...], preferred_element_type=jnp.float32)
    x = x + jnp.dot(se_ref[...], w1s_ref[...],
                    preferred_element_type=jnp.float32)
    h = jnp.maximum(x + b1_ref[...], 0.0)
    h = jnp.maximum(
        jnp.dot(h, w2_ref[...], preferred_element_type=jnp.float32)
        + b2_ref[...], 0.0)
    h = jnp.maximum(
        jnp.dot(h, w3_ref[...], preferred_element_type=jnp.float32)
        + b3_ref[...], 0.0)
    o = jnp.sum(h * wo_ref[...], axis=1, keepdims=True) + bo_ref[...]
    out_ref[...] = 1.0 / (1.0 + jnp.exp(-o))


def _mlp(ue, se, w1u, w1s, b1, w2t, b2, w3t, b3, wo_row, bo):
    bn = 2048
    grid = (_B // bn,)
    full = lambda shape: pl.BlockSpec(shape, lambda i: (0, 0))
    return pl.pallas_call(
        _mlp_body,
        grid=grid,
        in_specs=[
            pl.BlockSpec((bn, _E), lambda i: (i, 0)),
            pl.BlockSpec((bn, _E), lambda i: (i, 0)),
            full((_E, 128)),
            full((_E, 128)),
            full((1, 128)),
            full((128, 64)),
            full((1, 64)),
            full((64, 32)),
            full((1, 32)),
            full((1, 32)),
            full((1, 1)),
        ],
        out_specs=pl.BlockSpec((bn, 1), lambda i: (i, 0)),
        out_shape=jax.ShapeDtypeStruct((_B, 1), jnp.float32),
    )(ue, se, w1u, w1s, b1, w2t, b2, w3t, b3, wo_row, bo)


def kernel(user_ids, symbol_ids, user_table, symbol_table,
           W1, b1, W2, b2, W3, b3, Wo, bo):
    uids = user_ids.astype(jnp.int32)
    sids = symbol_ids.astype(jnp.int32)
    ut3 = user_table.reshape(-1, 8, _E)
    st3 = symbol_table.reshape(-1, 8, _E)
    ue, se = _gather(uids, sids, ut3, st3)
    w1t = W1.T
    return _mlp(ue, se, w1t[:_E], w1t[_E:], b1.reshape(1, -1),
                W2.T, b2.reshape(1, -1), W3.T, b3.reshape(1, -1),
                Wo.reshape(1, -1), bo.reshape(1, 1))
